# baseline (speedup 1.0000x reference)
import jax
import jax.numpy as jnp
from jax.experimental import pallas as pl
from jax.experimental.pallas import tpu as pltpu

_NUM_GROUPS = 16
_FEAT = 128
_WIN = 32
_OUT_ROWS = _NUM_GROUPS * _NUM_GROUPS


def _gmm_kernel(ind_ref, gl_hbm, right_hbm, out_ref,
                lhs_ref, right_ref, lsem, rsem):
    n_rows = gl_hbm.shape[0]
    rcp = pltpu.make_async_copy(right_hbm, right_ref, rsem)
    rcp.start()
    lcopies = []
    for i in range(_NUM_GROUPS):
        start = jnp.minimum(jnp.maximum(ind_ref[i, 0], 0), n_rows - _WIN)
        lcp = pltpu.make_async_copy(gl_hbm.at[pl.ds(start, _WIN), :],
                                    lhs_ref.at[i], lsem.at[i])
        lcp.start()
        lcopies.append(lcp)
    for i in range(_NUM_GROUPS):
        lcopies[i].wait()
    rcp.wait()
    for i in range(_NUM_GROUPS):
        cnt = 2 * i + 1
        res = jnp.dot(lhs_ref[i], right_ref[i],
                      preferred_element_type=jnp.float32)
        out_ref[i * i:i * i + cnt, :] = res[:cnt, :]


def kernel(grouped_left, right, ind_group):
    return pl.pallas_call(
        _gmm_kernel,
        in_specs=[
            pl.BlockSpec(memory_space=pltpu.SMEM),
            pl.BlockSpec(memory_space=pl.ANY),
            pl.BlockSpec(memory_space=pl.ANY),
        ],
        out_specs=pl.BlockSpec(memory_space=pltpu.VMEM),
        out_shape=jax.ShapeDtypeStruct((_OUT_ROWS, _FEAT), jnp.float32),
        scratch_shapes=[
            pltpu.VMEM((_NUM_GROUPS, _WIN, _FEAT), jnp.float32),
            pltpu.VMEM((_NUM_GROUPS, _FEAT, _FEAT), jnp.float32),
            pltpu.SemaphoreType.DMA((_NUM_GROUPS,)),
            pltpu.SemaphoreType.DMA,
        ],
    )(ind_group.astype(jnp.int32), grouped_left, right)


# speedup vs baseline: 1.0156x; 1.0156x over previous
import jax
import jax.numpy as jnp
from jax.experimental import pallas as pl
from jax.experimental.pallas import tpu as pltpu

_NUM_GROUPS = 16
_FEAT = 128
_WIN = 32
_OUT_ROWS = _NUM_GROUPS * _NUM_GROUPS


def _gmm_kernel(ind_hbm, gl_hbm, right_hbm, out_ref,
                ind_ref, lhs_ref, right_ref, isem, lsem, rsem):
    n_rows = gl_hbm.shape[0]
    rcp = pltpu.make_async_copy(right_hbm, right_ref, rsem)
    rcp.start()
    icp = pltpu.make_async_copy(ind_hbm, ind_ref, isem)
    icp.start()
    icp.wait()
    lcopies = []
    for i in range(_NUM_GROUPS):
        start = jnp.minimum(jnp.maximum(ind_ref[i, 0], 0), n_rows - _WIN)
        lcp = pltpu.make_async_copy(gl_hbm.at[pl.ds(start, _WIN), :],
                                    lhs_ref.at[i], lsem.at[i])
        lcp.start()
        lcopies.append(lcp)
    for i in range(_NUM_GROUPS):
        lcopies[i].wait()
    rcp.wait()
    for i in range(_NUM_GROUPS):
        cnt = 2 * i + 1
        res = jnp.dot(lhs_ref[i], right_ref[i],
                      preferred_element_type=jnp.float32)
        out_ref[i * i:i * i + cnt, :] = res[:cnt, :]


def kernel(grouped_left, right, ind_group):
    return pl.pallas_call(
        _gmm_kernel,
        in_specs=[
            pl.BlockSpec(memory_space=pl.ANY),
            pl.BlockSpec(memory_space=pl.ANY),
            pl.BlockSpec(memory_space=pl.ANY),
        ],
        out_specs=pl.BlockSpec(memory_space=pltpu.VMEM),
        out_shape=jax.ShapeDtypeStruct((_OUT_ROWS, _FEAT), jnp.float32),
        scratch_shapes=[
            pltpu.SMEM((_NUM_GROUPS, 2), jnp.int32),
            pltpu.VMEM((_NUM_GROUPS, _WIN, _FEAT), jnp.float32),
            pltpu.VMEM((_NUM_GROUPS, _FEAT, _FEAT), jnp.float32),
            pltpu.SemaphoreType.DMA,
            pltpu.SemaphoreType.DMA((_NUM_GROUPS,)),
            pltpu.SemaphoreType.DMA,
        ],
    )(ind_group.astype(jnp.int32), grouped_left, right)
